# Initial kernel scaffold; baseline (speedup 1.0000x reference)
import jax
import jax.numpy as jnp
from jax.experimental import pallas as pl


def kernel(feat, edge_index, Wc, Wp, attn_src, attn_dst, pos_attn_src, pos_attn_dst, att_combination):
    raise NotImplementedError("write your pallas kernel here")


# speedup vs baseline: 67.4854x; 67.4854x over previous
import functools

import jax
import jax.numpy as jnp
from jax import lax
from jax.experimental import pallas as pl
from jax.experimental.pallas import tpu as pltpu
from jax.experimental.pallas import tpu_sc as plsc

N = 10000
E = 320000
IN_DIM = 128
POS_DIM = 16
CONTENT_DIM = IN_DIM - POS_DIM
H = 4
D = 32
PD = D // 4
TW = 16

NC = 2
NS = 16
NW = NC * NS
EPW = E // NW
BLK = 80
NBLK = EPW // BLK
RPT = 624

RB = 2000
GRID_A = N // RB


def _prep_body(feat_ref, wct_ref, wpt_ref, w16_ref, hc_ref, t_ref):
    f = feat_ref[...]
    hc = jnp.dot(f, wct_ref[...], preferred_element_type=jnp.float32)
    hp = jnp.dot(f, wpt_ref[...], preferred_element_type=jnp.float32)
    hc_ref[...] = hc
    hchp = jnp.concatenate([hc, hp], axis=1)
    t_ref[...] = jnp.dot(hchp, w16_ref[...], preferred_element_type=jnp.float32)


def _prep(feat, wct_pad, wpt_pad, w16):
    return pl.pallas_call(
        _prep_body,
        grid=(GRID_A,),
        in_specs=[
            pl.BlockSpec((RB, IN_DIM), lambda i: (i, 0)),
            pl.BlockSpec((IN_DIM, IN_DIM), lambda i: (0, 0)),
            pl.BlockSpec((IN_DIM, H * PD), lambda i: (0, 0)),
            pl.BlockSpec((IN_DIM + H * PD, TW), lambda i: (0, 0)),
        ],
        out_specs=[
            pl.BlockSpec((RB, IN_DIM), lambda i: (i, 0)),
            pl.BlockSpec((RB, TW), lambda i: (i, 0)),
        ],
        out_shape=[
            jax.ShapeDtypeStruct((N, IN_DIM), jnp.float32),
            jax.ShapeDtypeStruct((N, TW), jnp.float32),
        ],
    )(feat, wct_pad, wpt_pad, w16)


_GDN = lax.GatherDimensionNumbers(
    offset_dims=(), collapsed_slice_dims=(0,), start_index_map=(0,))


def _take16(x, idx):
    return lax.gather(x, idx[:, None], _GDN, (1,),
                      mode=lax.GatherScatterMode.PROMISE_IN_BOUNDS)

def _edge_kernel(hc_hbm, t_hbm, ei_hbm, acc_out, den_out,
                 src_v, dst_v, sv, dv, wv, hc_v, acc_sh, den_sh,
                 sem1, sem2, sem3):
    c = lax.axis_index("c")
    s = lax.axis_index("s")
    wid = c * NS + s

    zero16 = jnp.zeros((16,), jnp.float32)
    lanes0 = jnp.arange(16, dtype=jnp.int32)
    dperm = lanes0 % H + H
    headmask = lanes0 < H

    def _zero_bufs(r, carry):
        for j in range(IN_DIM // 16):
            hc_v[r, pl.ds(16 * j, 16)] = zero16
        wv[r, pl.ds(0, 16)] = zero16
        return carry
    lax.fori_loop(0, BLK, _zero_bufs, 0)

    r0 = s * RPT
    nfull = RPT // BLK
    tail = RPT - nfull * BLK
    for k in range(nfull):
        pltpu.sync_copy(hc_v, acc_sh.at[pl.ds(r0 + k * BLK, BLK)])
        pltpu.sync_copy(wv, den_sh.at[pl.ds(r0 + k * BLK, BLK)])
    pltpu.sync_copy(hc_v.at[pl.ds(0, tail)], acc_sh.at[pl.ds(r0 + nfull * BLK, tail)])
    pltpu.sync_copy(wv.at[pl.ds(0, tail)], den_sh.at[pl.ds(r0 + nfull * BLK, tail)])

    @pl.when(s == NS - 1)
    def _zero_last():
        pltpu.sync_copy(hc_v.at[pl.ds(0, 16)], acc_sh.at[pl.ds(NS * RPT, 16)])
        pltpu.sync_copy(wv.at[pl.ds(0, 16)], den_sh.at[pl.ds(NS * RPT, 16)])

    plsc.subcore_barrier()

    def _block(blk, carry):
        e0 = wid * EPW + blk * BLK
        pltpu.sync_copy(ei_hbm.at[pl.ds(e0, BLK)], src_v)
        pltpu.sync_copy(ei_hbm.at[pl.ds(E + e0, BLK)], dst_v)
        cp1 = pltpu.async_copy(t_hbm.at[src_v], sv, sem1)
        cp2 = pltpu.async_copy(t_hbm.at[dst_v], dv, sem2)
        cp3 = pltpu.async_copy(hc_hbm.at[src_v], hc_v, sem3)
        cp1.wait()
        cp2.wait()
        cp3.wait()

        def _edge(e, carry2):
            a = sv[e, pl.ds(0, 16)]
            b = dv[e, pl.ds(0, 16)]
            x = a + _take16(b, dperm)
            x = jnp.where(x >= 0.0, x, 0.2 * x)
            w16 = jnp.exp(x)
            wv[e, pl.ds(0, 16)] = jnp.where(headmask, w16, 0.0)
            for h in range(H):
                wb = _take16(w16, jnp.full((16,), h, jnp.int32))
                for q in range(2):
                    c0 = h * D + q * 16
                    hc_v[e, pl.ds(c0, 16)] = hc_v[e, pl.ds(c0, 16)] * wb
            return carry2
        lax.fori_loop(0, BLK, _edge, 0)

        pltpu.sync_copy(hc_v, acc_sh.at[dst_v], add=True)
        pltpu.sync_copy(wv, den_sh.at[dst_v], add=True)
        return carry

    lax.fori_loop(0, NBLK, _block, 0)
    plsc.subcore_barrier()

    for k in range(nfull):
        pltpu.sync_copy(acc_sh.at[pl.ds(r0 + k * BLK, BLK)], hc_v)
        pltpu.sync_copy(hc_v, acc_out.at[c, pl.ds(r0 + k * BLK, BLK)])
        pltpu.sync_copy(den_sh.at[pl.ds(r0 + k * BLK, BLK)], wv)
        pltpu.sync_copy(wv, den_out.at[c, pl.ds(r0 + k * BLK, BLK)])
    pltpu.sync_copy(acc_sh.at[pl.ds(r0 + nfull * BLK, tail)], hc_v.at[pl.ds(0, tail)])
    pltpu.sync_copy(hc_v.at[pl.ds(0, tail)], acc_out.at[c, pl.ds(r0 + nfull * BLK, tail)])
    pltpu.sync_copy(den_sh.at[pl.ds(r0 + nfull * BLK, tail)], wv.at[pl.ds(0, tail)])
    pltpu.sync_copy(wv.at[pl.ds(0, tail)], den_out.at[c, pl.ds(r0 + nfull * BLK, tail)])

    @pl.when(s == NS - 1)
    def _drain_last():
        pltpu.sync_copy(acc_sh.at[pl.ds(NS * RPT, 16)], hc_v.at[pl.ds(0, 16)])
        pltpu.sync_copy(hc_v.at[pl.ds(0, 16)], acc_out.at[c, pl.ds(NS * RPT, 16)])
        pltpu.sync_copy(den_sh.at[pl.ds(NS * RPT, 16)], wv.at[pl.ds(0, 16)])
        pltpu.sync_copy(wv.at[pl.ds(0, 16)], den_out.at[c, pl.ds(NS * RPT, 16)])


def _edge_pass(hc, t_tab, edge_index):
    mesh = plsc.VectorSubcoreMesh(core_axis_name="c", subcore_axis_name="s")
    k = functools.partial(
        pl.kernel,
        mesh=mesh,
        compiler_params=pltpu.CompilerParams(use_tc_tiling_on_sc=False),
        out_type=[
            jax.ShapeDtypeStruct((NC, N, IN_DIM), jnp.float32),
            jax.ShapeDtypeStruct((NC, N, TW), jnp.float32),
        ],
        scratch_types=[
            pltpu.VMEM((BLK,), jnp.int32),
            pltpu.VMEM((BLK,), jnp.int32),
            pltpu.VMEM((BLK, TW), jnp.float32),
            pltpu.VMEM((BLK, TW), jnp.float32),
            pltpu.VMEM((BLK, TW), jnp.float32),
            pltpu.VMEM((BLK, IN_DIM), jnp.float32),
            pltpu.VMEM_SHARED((N, IN_DIM), jnp.float32),
            pltpu.VMEM_SHARED((N, TW), jnp.float32),
            pltpu.SemaphoreType.DMA,
            pltpu.SemaphoreType.DMA,
            pltpu.SemaphoreType.DMA,
        ],
    )(_edge_kernel)
    return k(hc, t_tab, edge_index.reshape(-1))


def _combine_body(acc_ref, den_ref, feat_ref, out_ref):
    a = acc_ref[0] + acc_ref[1]
    dn = den_ref[0, :, :H] + den_ref[1, :, :H]
    dnb = jnp.broadcast_to(dn[:, :, None], (RB, H, D)).reshape(RB, H * D)
    out_ref[...] = a / (dnb + 1e-9) + feat_ref[...]


def _combine(acc, den, feat):
    return pl.pallas_call(
        _combine_body,
        grid=(GRID_A,),
        in_specs=[
            pl.BlockSpec((NC, RB, IN_DIM), lambda i: (0, i, 0)),
            pl.BlockSpec((NC, RB, TW), lambda i: (0, i, 0)),
            pl.BlockSpec((RB, IN_DIM), lambda i: (i, 0)),
        ],
        out_specs=pl.BlockSpec((RB, IN_DIM), lambda i: (i, 0)),
        out_shape=jax.ShapeDtypeStruct((N, IN_DIM), jnp.float32),
    )(acc, den, feat)


def kernel(feat, edge_index, Wc, Wp, attn_src, attn_dst, pos_attn_src, pos_attn_dst, att_combination):
    f32 = jnp.float32
    c0 = att_combination[:, 0]
    c1 = att_combination[:, 1]
    wct_pad = jnp.zeros((IN_DIM, IN_DIM), f32).at[:CONTENT_DIM, :].set(Wc.T)
    wpt_pad = jnp.zeros((IN_DIM, H * PD), f32).at[CONTENT_DIM:, :].set(Wp.T)
    eye = jnp.eye(H, dtype=f32)
    sc = ((attn_src[0] * c0[:, None])[:, :, None] * eye[:, None, :]).reshape(H * D, H)
    dc = ((attn_dst[0] * c0[:, None])[:, :, None] * eye[:, None, :]).reshape(H * D, H)
    sp = ((pos_attn_src[0] * c1[:, None])[:, :, None] * eye[:, None, :]).reshape(H * PD, H)
    dp = ((pos_attn_dst[0] * c1[:, None])[:, :, None] * eye[:, None, :]).reshape(H * PD, H)
    w16 = jnp.concatenate(
        [jnp.concatenate([sc, dc], axis=1), jnp.concatenate([sp, dp], axis=1)], axis=0)
    w16 = jnp.pad(w16, ((0, 0), (0, TW - 2 * H)))

    hc, t_tab = _prep(feat, wct_pad, wpt_pad, w16)
    acc, den = _edge_pass(hc, t_tab, edge_index)
    return _combine(acc, den, feat)


# speedup vs baseline: 86.1446x; 1.2765x over previous
import functools

import jax
import jax.numpy as jnp
from jax import lax
from jax.experimental import pallas as pl
from jax.experimental.pallas import tpu as pltpu
from jax.experimental.pallas import tpu_sc as plsc

N = 10000
E = 320000
IN_DIM = 128
POS_DIM = 16
CONTENT_DIM = IN_DIM - POS_DIM
H = 4
D = 32
PD = D // 4
TW = 16

NC = 2
NS = 16
NW = NC * NS
EPW = E // NW
BLK = 80
NBLK = EPW // BLK
RPT = 624

RB = 2000
GRID_A = N // RB


def _prep_body(feat_ref, wct_ref, wpt_ref, w16_ref, hc_ref, t_ref):
    f = feat_ref[...]
    hc = jnp.dot(f, wct_ref[...], preferred_element_type=jnp.float32)
    hp = jnp.dot(f, wpt_ref[...], preferred_element_type=jnp.float32)
    hc_ref[...] = hc
    hchp = jnp.concatenate([hc, hp], axis=1)
    t_ref[...] = jnp.dot(hchp, w16_ref[...], preferred_element_type=jnp.float32)


def _prep(feat, wct_pad, wpt_pad, w16):
    return pl.pallas_call(
        _prep_body,
        grid=(GRID_A,),
        in_specs=[
            pl.BlockSpec((RB, IN_DIM), lambda i: (i, 0)),
            pl.BlockSpec((IN_DIM, IN_DIM), lambda i: (0, 0)),
            pl.BlockSpec((IN_DIM, H * PD), lambda i: (0, 0)),
            pl.BlockSpec((IN_DIM + H * PD, TW), lambda i: (0, 0)),
        ],
        out_specs=[
            pl.BlockSpec((RB, IN_DIM), lambda i: (i, 0)),
            pl.BlockSpec((RB, TW), lambda i: (i, 0)),
        ],
        out_shape=[
            jax.ShapeDtypeStruct((N, IN_DIM), jnp.float32),
            jax.ShapeDtypeStruct((N, TW), jnp.float32),
        ],
    )(feat, wct_pad, wpt_pad, w16)


_GDN = lax.GatherDimensionNumbers(
    offset_dims=(), collapsed_slice_dims=(0,), start_index_map=(0,))


def _take16(x, idx):
    return lax.gather(x, idx[:, None], _GDN, (1,),
                      mode=lax.GatherScatterMode.PROMISE_IN_BOUNDS)

def _edge_kernel(hc_hbm, t_hbm, ei_hbm, acc_out, den_out,
                 src_v, dst_v, sv, dv, wv, hc_v,
                 src_v1, dst_v1, sv1, dv1, hc_v1, acc_sh, den_sh,
                 sem1, sem2, sem3, sem4, sem5, sem6):
    c = lax.axis_index("c")
    s = lax.axis_index("s")
    wid = c * NS + s

    zero16 = jnp.zeros((16,), jnp.float32)
    lanes0 = jnp.arange(16, dtype=jnp.int32)
    dperm = lanes0 % H + H
    headmask = lanes0 < H

    def _zero_bufs(r, carry):
        for j in range(IN_DIM // 16):
            hc_v[r, pl.ds(16 * j, 16)] = zero16
        wv[r, pl.ds(0, 16)] = zero16
        return carry
    lax.fori_loop(0, BLK, _zero_bufs, 0)

    r0 = s * RPT
    nfull = RPT // BLK
    tail = RPT - nfull * BLK
    for k in range(nfull):
        pltpu.sync_copy(hc_v, acc_sh.at[pl.ds(r0 + k * BLK, BLK)])
        pltpu.sync_copy(wv, den_sh.at[pl.ds(r0 + k * BLK, BLK)])
    pltpu.sync_copy(hc_v.at[pl.ds(0, tail)], acc_sh.at[pl.ds(r0 + nfull * BLK, tail)])
    pltpu.sync_copy(wv.at[pl.ds(0, tail)], den_sh.at[pl.ds(r0 + nfull * BLK, tail)])

    @pl.when(s == NS - 1)
    def _zero_last():
        pltpu.sync_copy(hc_v.at[pl.ds(0, 16)], acc_sh.at[pl.ds(NS * RPT, 16)])
        pltpu.sync_copy(wv.at[pl.ds(0, 16)], den_sh.at[pl.ds(NS * RPT, 16)])

    plsc.subcore_barrier()

    bufs = ((src_v, dst_v, sv, dv, hc_v, sem1, sem2, sem3),
            (src_v1, dst_v1, sv1, dv1, hc_v1, sem4, sem5, sem6))

    def _issue(blk, buf):
        sr, ds_, svb, dvb, hcb, s1, s2, s3 = buf
        e0 = wid * EPW + blk * BLK
        pltpu.sync_copy(ei_hbm.at[pl.ds(e0, BLK)], sr)
        pltpu.sync_copy(ei_hbm.at[pl.ds(E + e0, BLK)], ds_)
        pltpu.async_copy(t_hbm.at[sr], svb, s1)
        pltpu.async_copy(t_hbm.at[ds_], dvb, s2)
        pltpu.async_copy(hc_hbm.at[sr], hcb, s3)

    def _wait(buf):
        sr, ds_, svb, dvb, hcb, s1, s2, s3 = buf
        pltpu.make_async_copy(t_hbm.at[sr], svb, s1).wait()
        pltpu.make_async_copy(t_hbm.at[ds_], dvb, s2).wait()
        pltpu.make_async_copy(hc_hbm.at[sr], hcb, s3).wait()

    def _compute_scatter(buf):
        sr, ds_, svb, dvb, hcb, s1, s2, s3 = buf

        def _edge(e, carry2):
            a = svb[e, pl.ds(0, 16)]
            b = dvb[e, pl.ds(0, 16)]
            x = a + _take16(b, dperm)
            x = jnp.where(x >= 0.0, x, 0.2 * x)
            w16 = jnp.exp(x)
            wv[e, pl.ds(0, 16)] = jnp.where(headmask, w16, 0.0)
            for h in range(H):
                wb = _take16(w16, jnp.full((16,), h, jnp.int32))
                for q in range(2):
                    c0 = h * D + q * 16
                    hcb[e, pl.ds(c0, 16)] = hcb[e, pl.ds(c0, 16)] * wb
            return carry2
        lax.fori_loop(0, BLK, _edge, 0)

        pltpu.sync_copy(hcb, acc_sh.at[ds_], add=True)
        pltpu.sync_copy(wv, den_sh.at[ds_], add=True)

    _issue(0, bufs[0])

    def _pair(i, carry):
        _issue(2 * i + 1, bufs[1])
        _wait(bufs[0])
        _compute_scatter(bufs[0])
        _issue(jnp.minimum(2 * i + 2, NBLK - 1), bufs[0])
        _wait(bufs[1])
        _compute_scatter(bufs[1])
        return carry

    lax.fori_loop(0, (NBLK - 1) // 2, _pair, 0)
    _wait(bufs[0])
    _compute_scatter(bufs[0])
    plsc.subcore_barrier()

    for k in range(nfull):
        pltpu.sync_copy(acc_sh.at[pl.ds(r0 + k * BLK, BLK)], hc_v)
        pltpu.sync_copy(hc_v, acc_out.at[c, pl.ds(r0 + k * BLK, BLK)])
        pltpu.sync_copy(den_sh.at[pl.ds(r0 + k * BLK, BLK)], wv)
        pltpu.sync_copy(wv, den_out.at[c, pl.ds(r0 + k * BLK, BLK)])
    pltpu.sync_copy(acc_sh.at[pl.ds(r0 + nfull * BLK, tail)], hc_v.at[pl.ds(0, tail)])
    pltpu.sync_copy(hc_v.at[pl.ds(0, tail)], acc_out.at[c, pl.ds(r0 + nfull * BLK, tail)])
    pltpu.sync_copy(den_sh.at[pl.ds(r0 + nfull * BLK, tail)], wv.at[pl.ds(0, tail)])
    pltpu.sync_copy(wv.at[pl.ds(0, tail)], den_out.at[c, pl.ds(r0 + nfull * BLK, tail)])

    @pl.when(s == NS - 1)
    def _drain_last():
        pltpu.sync_copy(acc_sh.at[pl.ds(NS * RPT, 16)], hc_v.at[pl.ds(0, 16)])
        pltpu.sync_copy(hc_v.at[pl.ds(0, 16)], acc_out.at[c, pl.ds(NS * RPT, 16)])
        pltpu.sync_copy(den_sh.at[pl.ds(NS * RPT, 16)], wv.at[pl.ds(0, 16)])
        pltpu.sync_copy(wv.at[pl.ds(0, 16)], den_out.at[c, pl.ds(NS * RPT, 16)])


def _edge_pass(hc, t_tab, edge_index):
    mesh = plsc.VectorSubcoreMesh(core_axis_name="c", subcore_axis_name="s")
    k = functools.partial(
        pl.kernel,
        mesh=mesh,
        compiler_params=pltpu.CompilerParams(use_tc_tiling_on_sc=False),
        out_type=[
            jax.ShapeDtypeStruct((NC, N, IN_DIM), jnp.float32),
            jax.ShapeDtypeStruct((NC, N, TW), jnp.float32),
        ],
        scratch_types=[
            pltpu.VMEM((BLK,), jnp.int32),
            pltpu.VMEM((BLK,), jnp.int32),
            pltpu.VMEM((BLK, TW), jnp.float32),
            pltpu.VMEM((BLK, TW), jnp.float32),
            pltpu.VMEM((BLK, TW), jnp.float32),
            pltpu.VMEM((BLK, IN_DIM), jnp.float32),
            pltpu.VMEM((BLK,), jnp.int32),
            pltpu.VMEM((BLK,), jnp.int32),
            pltpu.VMEM((BLK, TW), jnp.float32),
            pltpu.VMEM((BLK, TW), jnp.float32),
            pltpu.VMEM((BLK, IN_DIM), jnp.float32),
            pltpu.VMEM_SHARED((N, IN_DIM), jnp.float32),
            pltpu.VMEM_SHARED((N, TW), jnp.float32),
            pltpu.SemaphoreType.DMA,
            pltpu.SemaphoreType.DMA,
            pltpu.SemaphoreType.DMA,
            pltpu.SemaphoreType.DMA,
            pltpu.SemaphoreType.DMA,
            pltpu.SemaphoreType.DMA,
        ],
    )(_edge_kernel)
    return k(hc, t_tab, edge_index.reshape(-1))


def _combine_body(acc_ref, den_ref, feat_ref, out_ref):
    a = acc_ref[0] + acc_ref[1]
    dn = den_ref[0, :, :H] + den_ref[1, :, :H]
    dnb = jnp.broadcast_to(dn[:, :, None], (RB, H, D)).reshape(RB, H * D)
    out_ref[...] = a / (dnb + 1e-9) + feat_ref[...]


def _combine(acc, den, feat):
    return pl.pallas_call(
        _combine_body,
        grid=(GRID_A,),
        in_specs=[
            pl.BlockSpec((NC, RB, IN_DIM), lambda i: (0, i, 0)),
            pl.BlockSpec((NC, RB, TW), lambda i: (0, i, 0)),
            pl.BlockSpec((RB, IN_DIM), lambda i: (i, 0)),
        ],
        out_specs=pl.BlockSpec((RB, IN_DIM), lambda i: (i, 0)),
        out_shape=jax.ShapeDtypeStruct((N, IN_DIM), jnp.float32),
    )(acc, den, feat)


def kernel(feat, edge_index, Wc, Wp, attn_src, attn_dst, pos_attn_src, pos_attn_dst, att_combination):
    f32 = jnp.float32
    c0 = att_combination[:, 0]
    c1 = att_combination[:, 1]
    wct_pad = jnp.zeros((IN_DIM, IN_DIM), f32).at[:CONTENT_DIM, :].set(Wc.T)
    wpt_pad = jnp.zeros((IN_DIM, H * PD), f32).at[CONTENT_DIM:, :].set(Wp.T)
    eye = jnp.eye(H, dtype=f32)
    sc = ((attn_src[0] * c0[:, None])[:, :, None] * eye[:, None, :]).reshape(H * D, H)
    dc = ((attn_dst[0] * c0[:, None])[:, :, None] * eye[:, None, :]).reshape(H * D, H)
    sp = ((pos_attn_src[0] * c1[:, None])[:, :, None] * eye[:, None, :]).reshape(H * PD, H)
    dp = ((pos_attn_dst[0] * c1[:, None])[:, :, None] * eye[:, None, :]).reshape(H * PD, H)
    w16 = jnp.concatenate(
        [jnp.concatenate([sc, dc], axis=1), jnp.concatenate([sp, dp], axis=1)], axis=0)
    w16 = jnp.pad(w16, ((0, 0), (0, TW - 2 * H)))

    hc, t_tab = _prep(feat, wct_pad, wpt_pad, w16)
    acc, den = _edge_pass(hc, t_tab, edge_index)
    return _combine(acc, den, feat)


# speedup vs baseline: 133.4530x; 1.5492x over previous
import functools

import jax
import jax.numpy as jnp
from jax import lax
from jax.experimental import pallas as pl
from jax.experimental.pallas import tpu as pltpu
from jax.experimental.pallas import tpu_sc as plsc

N = 10000
E = 320000
IN_DIM = 128
POS_DIM = 16
CONTENT_DIM = IN_DIM - POS_DIM
H = 4
D = 32
PD = D // 4
TW = 16

NC = 2
NS = 16
NW = NC * NS
EPW = E // NW
BLK = 80
NBLK = EPW // BLK
RPT = 624

RB = 2000
GRID_A = N // RB


def _prep_body(feat_ref, wct_ref, wpt_ref, w16_ref, hc_ref, t_ref):
    f = feat_ref[...]
    hc = jnp.dot(f, wct_ref[...], preferred_element_type=jnp.float32)
    hp = jnp.dot(f, wpt_ref[...], preferred_element_type=jnp.float32)
    hc_ref[...] = hc
    hchp = jnp.concatenate([hc, hp], axis=1)
    t_ref[...] = jnp.dot(hchp, w16_ref[...], preferred_element_type=jnp.float32)


def _prep(feat, wct_pad, wpt_pad, w16):
    return pl.pallas_call(
        _prep_body,
        grid=(GRID_A,),
        in_specs=[
            pl.BlockSpec((RB, IN_DIM), lambda i: (i, 0)),
            pl.BlockSpec((IN_DIM, IN_DIM), lambda i: (0, 0)),
            pl.BlockSpec((IN_DIM, H * PD), lambda i: (0, 0)),
            pl.BlockSpec((IN_DIM + H * PD, TW), lambda i: (0, 0)),
        ],
        out_specs=[
            pl.BlockSpec((RB, IN_DIM), lambda i: (i, 0)),
            pl.BlockSpec((RB, TW), lambda i: (i, 0)),
        ],
        out_shape=[
            jax.ShapeDtypeStruct((N, IN_DIM), jnp.float32),
            jax.ShapeDtypeStruct((N, TW), jnp.float32),
        ],
    )(feat, wct_pad, wpt_pad, w16)


_GDN = lax.GatherDimensionNumbers(
    offset_dims=(), collapsed_slice_dims=(0,), start_index_map=(0,))


def _take16(x, idx):
    return lax.gather(x, idx[:, None], _GDN, (1,),
                      mode=lax.GatherScatterMode.PROMISE_IN_BOUNDS)

def _edge_kernel(hc_hbm, t_hbm, ei_hbm, acc_out, den_out,
                 src_v, dst_v, sv, dv, wv, hc_v,
                 src_v1, dst_v1, sv1, dv1, hc_v1, acc_sh, den_sh,
                 sem1, sem2, sem3, sem4, sem5, sem6):
    c = lax.axis_index("c")
    s = lax.axis_index("s")
    wid = c * NS + s

    zero16 = jnp.zeros((16,), jnp.float32)
    lanes0 = jnp.arange(16, dtype=jnp.int32)
    dperm = lanes0 % H + H
    headmask = lanes0 < H

    def _zero_bufs(r, carry):
        for j in range(IN_DIM // 16):
            hc_v[r, pl.ds(16 * j, 16)] = zero16
        wv[r, pl.ds(0, 16)] = zero16
        return carry
    lax.fori_loop(0, BLK, _zero_bufs, 0)

    r0 = s * RPT
    nfull = RPT // BLK
    tail = RPT - nfull * BLK
    for k in range(nfull):
        pltpu.sync_copy(hc_v, acc_sh.at[pl.ds(r0 + k * BLK, BLK)])
        pltpu.sync_copy(wv, den_sh.at[pl.ds(r0 + k * BLK, BLK)])
    pltpu.sync_copy(hc_v.at[pl.ds(0, tail)], acc_sh.at[pl.ds(r0 + nfull * BLK, tail)])
    pltpu.sync_copy(wv.at[pl.ds(0, tail)], den_sh.at[pl.ds(r0 + nfull * BLK, tail)])

    @pl.when(s == NS - 1)
    def _zero_last():
        pltpu.sync_copy(hc_v.at[pl.ds(0, 16)], acc_sh.at[pl.ds(NS * RPT, 16)])
        pltpu.sync_copy(wv.at[pl.ds(0, 16)], den_sh.at[pl.ds(NS * RPT, 16)])

    plsc.subcore_barrier()

    bufs = ((src_v, dst_v, sv, dv, hc_v, sem1, sem2, sem3),
            (src_v1, dst_v1, sv1, dv1, hc_v1, sem4, sem5, sem6))

    def _issue(blk, buf):
        sr, ds_, svb, dvb, hcb, s1, s2, s3 = buf
        e0 = wid * EPW + blk * BLK
        pltpu.sync_copy(ei_hbm.at[pl.ds(e0, BLK)], sr)
        pltpu.sync_copy(ei_hbm.at[pl.ds(E + e0, BLK)], ds_)
        pltpu.async_copy(t_hbm.at[sr], svb, s1)
        pltpu.async_copy(t_hbm.at[ds_], dvb, s2)
        pltpu.async_copy(hc_hbm.at[sr], hcb, s3)

    def _wait(buf):
        sr, ds_, svb, dvb, hcb, s1, s2, s3 = buf
        pltpu.make_async_copy(t_hbm.at[sr], svb, s1).wait()
        pltpu.make_async_copy(t_hbm.at[ds_], dvb, s2).wait()
        pltpu.make_async_copy(hc_hbm.at[sr], hcb, s3).wait()

    def _compute_scatter(buf):
        sr, ds_, svb, dvb, hcb, s1, s2, s3 = buf

        @plsc.parallel_loop(0, BLK, 1, unroll=4)
        def _edge(e):
            a = svb[e, pl.ds(0, 16)]
            b = dvb[e, pl.ds(0, 16)]
            x = a + _take16(b, dperm)
            x = jnp.where(x >= 0.0, x, 0.2 * x)
            w16 = jnp.exp(x)
            wv[e, pl.ds(0, 16)] = jnp.where(headmask, w16, 0.0)
            for h in range(H):
                wb = _take16(w16, jnp.full((16,), h, jnp.int32))
                for q in range(2):
                    c0 = h * D + q * 16
                    hcb[e, pl.ds(c0, 16)] = hcb[e, pl.ds(c0, 16)] * wb

        pltpu.sync_copy(hcb, acc_sh.at[ds_], add=True)
        pltpu.sync_copy(wv, den_sh.at[ds_], add=True)

    _issue(0, bufs[0])

    def _pair(i, carry):
        _issue(2 * i + 1, bufs[1])
        _wait(bufs[0])
        _compute_scatter(bufs[0])
        _issue(jnp.minimum(2 * i + 2, NBLK - 1), bufs[0])
        _wait(bufs[1])
        _compute_scatter(bufs[1])
        return carry

    lax.fori_loop(0, (NBLK - 1) // 2, _pair, 0)
    _wait(bufs[0])
    _compute_scatter(bufs[0])
    plsc.subcore_barrier()

    for k in range(nfull):
        pltpu.sync_copy(acc_sh.at[pl.ds(r0 + k * BLK, BLK)], hc_v)
        pltpu.sync_copy(hc_v, acc_out.at[c, pl.ds(r0 + k * BLK, BLK)])
        pltpu.sync_copy(den_sh.at[pl.ds(r0 + k * BLK, BLK)], wv)
        pltpu.sync_copy(wv, den_out.at[c, pl.ds(r0 + k * BLK, BLK)])
    pltpu.sync_copy(acc_sh.at[pl.ds(r0 + nfull * BLK, tail)], hc_v.at[pl.ds(0, tail)])
    pltpu.sync_copy(hc_v.at[pl.ds(0, tail)], acc_out.at[c, pl.ds(r0 + nfull * BLK, tail)])
    pltpu.sync_copy(den_sh.at[pl.ds(r0 + nfull * BLK, tail)], wv.at[pl.ds(0, tail)])
    pltpu.sync_copy(wv.at[pl.ds(0, tail)], den_out.at[c, pl.ds(r0 + nfull * BLK, tail)])

    @pl.when(s == NS - 1)
    def _drain_last():
        pltpu.sync_copy(acc_sh.at[pl.ds(NS * RPT, 16)], hc_v.at[pl.ds(0, 16)])
        pltpu.sync_copy(hc_v.at[pl.ds(0, 16)], acc_out.at[c, pl.ds(NS * RPT, 16)])
        pltpu.sync_copy(den_sh.at[pl.ds(NS * RPT, 16)], wv.at[pl.ds(0, 16)])
        pltpu.sync_copy(wv.at[pl.ds(0, 16)], den_out.at[c, pl.ds(NS * RPT, 16)])


def _edge_pass(hc, t_tab, edge_index):
    mesh = plsc.VectorSubcoreMesh(core_axis_name="c", subcore_axis_name="s")
    k = functools.partial(
        pl.kernel,
        mesh=mesh,
        compiler_params=pltpu.CompilerParams(use_tc_tiling_on_sc=False),
        out_type=[
            jax.ShapeDtypeStruct((NC, N, IN_DIM), jnp.float32),
            jax.ShapeDtypeStruct((NC, N, TW), jnp.float32),
        ],
        scratch_types=[
            pltpu.VMEM((BLK,), jnp.int32),
            pltpu.VMEM((BLK,), jnp.int32),
            pltpu.VMEM((BLK, TW), jnp.float32),
            pltpu.VMEM((BLK, TW), jnp.float32),
            pltpu.VMEM((BLK, TW), jnp.float32),
            pltpu.VMEM((BLK, IN_DIM), jnp.float32),
            pltpu.VMEM((BLK,), jnp.int32),
            pltpu.VMEM((BLK,), jnp.int32),
            pltpu.VMEM((BLK, TW), jnp.float32),
            pltpu.VMEM((BLK, TW), jnp.float32),
            pltpu.VMEM((BLK, IN_DIM), jnp.float32),
            pltpu.VMEM_SHARED((N, IN_DIM), jnp.float32),
            pltpu.VMEM_SHARED((N, TW), jnp.float32),
            pltpu.SemaphoreType.DMA,
            pltpu.SemaphoreType.DMA,
            pltpu.SemaphoreType.DMA,
            pltpu.SemaphoreType.DMA,
            pltpu.SemaphoreType.DMA,
            pltpu.SemaphoreType.DMA,
        ],
    )(_edge_kernel)
    return k(hc, t_tab, edge_index.reshape(-1))


def _combine_body(acc_ref, den_ref, feat_ref, out_ref):
    a = acc_ref[0] + acc_ref[1]
    dn = den_ref[0, :, :H] + den_ref[1, :, :H]
    dnb = jnp.broadcast_to(dn[:, :, None], (RB, H, D)).reshape(RB, H * D)
    out_ref[...] = a / (dnb + 1e-9) + feat_ref[...]


def _combine(acc, den, feat):
    return pl.pallas_call(
        _combine_body,
        grid=(GRID_A,),
        in_specs=[
            pl.BlockSpec((NC, RB, IN_DIM), lambda i: (0, i, 0)),
            pl.BlockSpec((NC, RB, TW), lambda i: (0, i, 0)),
            pl.BlockSpec((RB, IN_DIM), lambda i: (i, 0)),
        ],
        out_specs=pl.BlockSpec((RB, IN_DIM), lambda i: (i, 0)),
        out_shape=jax.ShapeDtypeStruct((N, IN_DIM), jnp.float32),
    )(acc, den, feat)


def kernel(feat, edge_index, Wc, Wp, attn_src, attn_dst, pos_attn_src, pos_attn_dst, att_combination):
    f32 = jnp.float32
    c0 = att_combination[:, 0]
    c1 = att_combination[:, 1]
    wct_pad = jnp.zeros((IN_DIM, IN_DIM), f32).at[:CONTENT_DIM, :].set(Wc.T)
    wpt_pad = jnp.zeros((IN_DIM, H * PD), f32).at[CONTENT_DIM:, :].set(Wp.T)
    eye = jnp.eye(H, dtype=f32)
    sc = ((attn_src[0] * c0[:, None])[:, :, None] * eye[:, None, :]).reshape(H * D, H)
    dc = ((attn_dst[0] * c0[:, None])[:, :, None] * eye[:, None, :]).reshape(H * D, H)
    sp = ((pos_attn_src[0] * c1[:, None])[:, :, None] * eye[:, None, :]).reshape(H * PD, H)
    dp = ((pos_attn_dst[0] * c1[:, None])[:, :, None] * eye[:, None, :]).reshape(H * PD, H)
    w16 = jnp.concatenate(
        [jnp.concatenate([sc, dc], axis=1), jnp.concatenate([sp, dp], axis=1)], axis=0)
    w16 = jnp.pad(w16, ((0, 0), (0, TW - 2 * H)))

    hc, t_tab = _prep(feat, wct_pad, wpt_pad, w16)
    acc, den = _edge_pass(hc, t_tab, edge_index)
    return _combine(acc, den, feat)
